# baseline (device time: 93847 ns/iter reference)
import jax
import jax.numpy as jnp
from jax import lax
from jax.experimental import pallas as pl
from jax.experimental.pallas import tpu as pltpu


def kernel(x, A, B, C):
    Bsz, S, D = x.shape
    N = A.shape[-1]

    def body(x_ref, A_ref, B_ref, C_ref, out_ref, comm_ref, send_sem, recv_sem):
        my_x = lax.axis_index("x")
        my_y = lax.axis_index("y")
        peer = (1 - my_x, my_y)

        barrier = pltpu.get_barrier_semaphore()
        pl.semaphore_signal(barrier, inc=1, device_id=peer,
                            device_id_type=pl.DeviceIdType.MESH)
        pl.semaphore_wait(barrier, 1)

        dA = jnp.exp(A_ref[...])
        dAT = jnp.transpose(dA)[None, :, :]

        rdma = pltpu.make_async_remote_copy(
            src_ref=comm_ref,
            dst_ref=comm_ref,
            send_sem=send_sem,
            recv_sem=recv_sem,
            device_id=peer,
            device_id_type=pl.DeviceIdType.MESH,
        )

        @pl.when(my_x == 1)
        def _():
            rdma.wait_recv()

        h0 = jnp.where(my_x == 1, comm_ref[...],
                       jnp.zeros((Bsz, N, D), jnp.float32))

        def step(t, h):
            xt = x_ref[:, pl.ds(t, 1), :]
            bt = jnp.transpose(B_ref[:, pl.ds(t, 1), :], (0, 2, 1))
            ct = jnp.transpose(C_ref[:, pl.ds(t, 1), :], (0, 2, 1))
            h = h * dAT + bt * xt
            out_ref[:, pl.ds(t, 1), :] = jnp.sum(h * ct, axis=1,
                                                 keepdims=True)
            return h

        h_final = lax.fori_loop(0, S, step, h0)

        @pl.when(my_x == 0)
        def _():
            comm_ref[...] = h_final
            rdma.start()
            rdma.wait_send()

    return pl.pallas_call(
        body,
        out_shape=jax.ShapeDtypeStruct((Bsz, S, D), jnp.float32),
        in_specs=[pl.BlockSpec(memory_space=pltpu.VMEM)] * 4,
        out_specs=pl.BlockSpec(memory_space=pltpu.VMEM),
        scratch_shapes=[
            pltpu.VMEM((Bsz, N, D), jnp.float32),
            pltpu.SemaphoreType.DMA,
            pltpu.SemaphoreType.DMA,
        ],
        compiler_params=pltpu.CompilerParams(collective_id=0),
    )(x, A, B, C)


# device time: 53510 ns/iter; 1.7538x vs baseline; 1.7538x over previous
import jax
import jax.numpy as jnp
from jax import lax
from jax.experimental import pallas as pl
from jax.experimental.pallas import tpu as pltpu


def kernel(x, A, B, C):
    Bsz, S, D = x.shape
    N = A.shape[-1]

    def body(x_ref, A_ref, B_ref, C_ref, out_ref, comm_ref, send_sem, recv_sem):
        my_x = lax.axis_index("x")
        my_y = lax.axis_index("y")
        peer = (1 - my_x, my_y)

        barrier = pltpu.get_barrier_semaphore()
        pl.semaphore_signal(barrier, inc=1, device_id=peer,
                            device_id_type=pl.DeviceIdType.MESH)
        pl.semaphore_wait(barrier, 1)

        dA = jnp.exp(A_ref[...])
        dAT = jnp.transpose(dA)[None, :, :]

        rdma = pltpu.make_async_remote_copy(
            src_ref=comm_ref,
            dst_ref=comm_ref,
            send_sem=send_sem,
            recv_sem=recv_sem,
            device_id=peer,
            device_id_type=pl.DeviceIdType.MESH,
        )

        h0 = jnp.zeros((Bsz, N, D), jnp.float32)

        def step(t, h):
            xt = x_ref[:, pl.ds(t, 1), :]
            bt = jnp.transpose(B_ref[:, pl.ds(t, 1), :], (0, 2, 1))
            ct = jnp.transpose(C_ref[:, pl.ds(t, 1), :], (0, 2, 1))
            h = h * dAT + bt * xt
            out_ref[:, pl.ds(t, 1), :] = jnp.sum(h * ct, axis=1,
                                                 keepdims=True)
            return h

        h_final = lax.fori_loop(0, S, step, h0)

        @pl.when(my_x == 0)
        def _():
            comm_ref[...] = h_final
            rdma.start()
            rdma.wait_send()

        @pl.when(my_x == 1)
        def _():
            rdma.wait_recv()
            AT = jnp.transpose(A_ref[...])
            tp1 = (lax.broadcasted_iota(jnp.int32, (S, 1), 0) + 1
                   ).astype(jnp.float32)
            acc = out_ref[...]
            for n in range(N):
                En = jnp.exp(tp1 * AT[n][None, :])
                cn = C_ref[:, :, n:n + 1]
                hn = comm_ref[:, n:n + 1, :]
                acc = acc + cn * (En[None, :, :] * hn)
            out_ref[...] = acc

    return pl.pallas_call(
        body,
        out_shape=jax.ShapeDtypeStruct((Bsz, S, D), jnp.float32),
        in_specs=[pl.BlockSpec(memory_space=pltpu.VMEM)] * 4,
        out_specs=pl.BlockSpec(memory_space=pltpu.VMEM),
        scratch_shapes=[
            pltpu.VMEM((Bsz, N, D), jnp.float32),
            pltpu.SemaphoreType.DMA,
            pltpu.SemaphoreType.DMA,
        ],
        compiler_params=pltpu.CompilerParams(collective_id=0),
    )(x, A, B, C)


# device time: 19742 ns/iter; 4.7537x vs baseline; 2.7105x over previous
import jax
import jax.numpy as jnp
from jax import lax
from jax.experimental import pallas as pl
from jax.experimental.pallas import tpu as pltpu


def kernel(x, A, B, C):
    Bsz, S, D = x.shape
    N = A.shape[-1]

    def body(x_ref, A_ref, B_ref, C_ref, out_ref, comm_ref, send_sem, recv_sem):
        my_x = lax.axis_index("x")
        my_y = lax.axis_index("y")
        peer = (1 - my_x, my_y)

        barrier = pltpu.get_barrier_semaphore()
        pl.semaphore_signal(barrier, inc=1, device_id=peer,
                            device_id_type=pl.DeviceIdType.MESH)
        pl.semaphore_wait(barrier, 1)

        dA = jnp.exp(A_ref[...])
        dAT = jnp.transpose(dA)[None, :, :]

        rdma = pltpu.make_async_remote_copy(
            src_ref=comm_ref,
            dst_ref=comm_ref,
            send_sem=send_sem,
            recv_sem=recv_sem,
            device_id=peer,
            device_id_type=pl.DeviceIdType.MESH,
        )

        h0 = jnp.zeros((Bsz, N, D), jnp.float32)

        L = 8

        def blk(k, h):
            t0 = k * L
            xb = x_ref[:, pl.ds(t0, L), :]
            bb = jnp.transpose(B_ref[:, pl.ds(t0, L), :], (0, 2, 1))
            cb = jnp.transpose(C_ref[:, pl.ds(t0, L), :], (0, 2, 1))
            ys = []
            for j in range(L):
                h = h * dAT + bb[:, :, j:j + 1] * xb[:, j:j + 1, :]
                ys.append(jnp.sum(h * cb[:, :, j:j + 1], axis=1,
                                  keepdims=True))
            out_ref[:, pl.ds(t0, L), :] = jnp.concatenate(ys, axis=1)
            return h

        h_final = lax.fori_loop(0, S // L, blk, h0)

        @pl.when(my_x == 0)
        def _():
            comm_ref[...] = h_final
            rdma.start()
            rdma.wait_send()

        @pl.when(my_x == 1)
        def _():
            rdma.wait_recv()
            AT = jnp.transpose(A_ref[...])
            tp1 = (lax.broadcasted_iota(jnp.int32, (S, 1), 0) + 1
                   ).astype(jnp.float32)
            acc = out_ref[...]
            for n in range(N):
                En = jnp.exp(tp1 * AT[n][None, :])
                cn = C_ref[:, :, n:n + 1]
                hn = comm_ref[:, n:n + 1, :]
                acc = acc + cn * (En[None, :, :] * hn)
            out_ref[...] = acc

    return pl.pallas_call(
        body,
        out_shape=jax.ShapeDtypeStruct((Bsz, S, D), jnp.float32),
        in_specs=[pl.BlockSpec(memory_space=pltpu.VMEM)] * 4,
        out_specs=pl.BlockSpec(memory_space=pltpu.VMEM),
        scratch_shapes=[
            pltpu.VMEM((Bsz, N, D), jnp.float32),
            pltpu.SemaphoreType.DMA,
            pltpu.SemaphoreType.DMA,
        ],
        compiler_params=pltpu.CompilerParams(collective_id=0),
    )(x, A, B, C)


# device time: 18075 ns/iter; 5.1921x vs baseline; 1.0922x over previous
import jax
import jax.numpy as jnp
from jax import lax
from jax.experimental import pallas as pl
from jax.experimental.pallas import tpu as pltpu


def kernel(x, A, B, C):
    Bsz, S, D = x.shape
    N = A.shape[-1]

    def body(x_ref, A_ref, B_ref, C_ref, out_ref, comm_ref, send_sem, recv_sem):
        my_x = lax.axis_index("x")
        my_y = lax.axis_index("y")
        peer = (1 - my_x, my_y)

        barrier = pltpu.get_barrier_semaphore()
        pl.semaphore_signal(barrier, inc=1, device_id=peer,
                            device_id_type=pl.DeviceIdType.MESH)
        pl.semaphore_wait(barrier, 1)

        dA = jnp.exp(A_ref[...])
        dAT = jnp.transpose(dA)[None, :, :]

        rdma = pltpu.make_async_remote_copy(
            src_ref=comm_ref,
            dst_ref=comm_ref,
            send_sem=send_sem,
            recv_sem=recv_sem,
            device_id=peer,
            device_id_type=pl.DeviceIdType.MESH,
        )

        h0 = jnp.zeros((Bsz, N, D), jnp.float32)

        L = 16

        h = h0
        for k in range(S // L):
            t0 = k * L
            xb = x_ref[:, t0:t0 + L, :]
            bb = jnp.transpose(B_ref[:, t0:t0 + L, :], (0, 2, 1))
            cb = jnp.transpose(C_ref[:, t0:t0 + L, :], (0, 2, 1))
            ys = []
            for j in range(L):
                h = h * dAT + bb[:, :, j:j + 1] * xb[:, j:j + 1, :]
                ys.append(jnp.sum(h * cb[:, :, j:j + 1], axis=1,
                                  keepdims=True))
            out_ref[:, t0:t0 + L, :] = jnp.concatenate(ys, axis=1)
        h_final = h

        @pl.when(my_x == 0)
        def _():
            comm_ref[...] = h_final
            rdma.start()
            rdma.wait_send()

        @pl.when(my_x == 1)
        def _():
            rdma.wait_recv()
            AT = jnp.transpose(A_ref[...])
            tp1 = (lax.broadcasted_iota(jnp.int32, (S, 1), 0) + 1
                   ).astype(jnp.float32)
            acc = out_ref[...]
            for n in range(N):
                En = jnp.exp(tp1 * AT[n][None, :])
                cn = C_ref[:, :, n:n + 1]
                hn = comm_ref[:, n:n + 1, :]
                acc = acc + cn * (En[None, :, :] * hn)
            out_ref[...] = acc

    return pl.pallas_call(
        body,
        out_shape=jax.ShapeDtypeStruct((Bsz, S, D), jnp.float32),
        in_specs=[pl.BlockSpec(memory_space=pltpu.VMEM)] * 4,
        out_specs=pl.BlockSpec(memory_space=pltpu.VMEM),
        scratch_shapes=[
            pltpu.VMEM((Bsz, N, D), jnp.float32),
            pltpu.SemaphoreType.DMA,
            pltpu.SemaphoreType.DMA,
        ],
        compiler_params=pltpu.CompilerParams(collective_id=0),
    )(x, A, B, C)


# device time: 10766 ns/iter; 8.7170x vs baseline; 1.6789x over previous
import jax
import jax.numpy as jnp
from jax import lax
from jax.experimental import pallas as pl
from jax.experimental.pallas import tpu as pltpu


def kernel(x, A, B, C):
    Bsz, S, D = x.shape
    N = A.shape[-1]

    def body(x_ref, A_ref, B_ref, C_ref, out_ref, comm_ref, send_sem, recv_sem):
        my_x = lax.axis_index("x")
        my_y = lax.axis_index("y")
        peer = (1 - my_x, my_y)

        barrier = pltpu.get_barrier_semaphore()
        pl.semaphore_signal(barrier, inc=1, device_id=peer,
                            device_id_type=pl.DeviceIdType.MESH)
        pl.semaphore_wait(barrier, 1)

        dA = jnp.exp(A_ref[...])
        dAT = jnp.transpose(dA)[None, :, :].astype(jnp.bfloat16)

        rdma = pltpu.make_async_remote_copy(
            src_ref=comm_ref,
            dst_ref=comm_ref,
            send_sem=send_sem,
            recv_sem=recv_sem,
            device_id=peer,
            device_id_type=pl.DeviceIdType.MESH,
        )

        h = jnp.zeros((Bsz, N, D), jnp.bfloat16)

        L = 32
        dn = (((1,), (1,)), ((0,), (0,)))

        for k in range(S // L):
            t0 = k * L
            xb = x_ref[:, t0:t0 + L, :].astype(jnp.bfloat16)
            bb = jnp.transpose(B_ref[:, t0:t0 + L, :],
                               (0, 2, 1)).astype(jnp.bfloat16)
            cb = C_ref[:, t0:t0 + L, :].astype(jnp.bfloat16)
            ys = []
            for j in range(L):
                h = h * dAT + bb[:, :, j:j + 1] * xb[:, j:j + 1, :]
                yt = lax.dot_general(cb[:, j, :], h, dn,
                                     preferred_element_type=jnp.float32)
                ys.append(yt[:, None, :])
            out_ref[:, t0:t0 + L, :] = jnp.concatenate(ys, axis=1)
        h_final = h

        @pl.when(my_x == 0)
        def _():
            comm_ref[...] = h_final.astype(jnp.float32)
            rdma.start()
            rdma.wait_send()

        @pl.when(my_x == 1)
        def _():
            AT = jnp.transpose(A_ref[...])
            q = S // 4
            tp1 = (lax.broadcasted_iota(jnp.int32, (q, 1), 0) + 1
                   ).astype(jnp.float32)
            Es = []
            for n in range(N):
                a_n = AT[n][None, :]
                Eq = jnp.exp(tp1 * a_n)
                pq = jnp.exp(float(q) * a_n)
                p2q = pq * pq
                Es.append(jnp.concatenate(
                    [Eq, Eq * pq, Eq * p2q, Eq * (p2q * pq)],
                    axis=0).astype(jnp.bfloat16))
            rdma.wait_recv()
            acc = out_ref[...].astype(jnp.bfloat16)
            for n in range(N):
                cn = C_ref[:, :, n:n + 1].astype(jnp.bfloat16)
                hn = comm_ref[:, n:n + 1, :].astype(jnp.bfloat16)
                acc = acc + cn * (Es[n][None, :, :] * hn)
            out_ref[...] = acc.astype(jnp.float32)

    return pl.pallas_call(
        body,
        out_shape=jax.ShapeDtypeStruct((Bsz, S, D), jnp.float32),
        in_specs=[pl.BlockSpec(memory_space=pltpu.VMEM)] * 4,
        out_specs=pl.BlockSpec(memory_space=pltpu.VMEM),
        scratch_shapes=[
            pltpu.VMEM((Bsz, N, D), jnp.float32),
            pltpu.SemaphoreType.DMA,
            pltpu.SemaphoreType.DMA,
        ],
        compiler_params=pltpu.CompilerParams(collective_id=0),
    )(x, A, B, C)
